# baseline (speedup 1.0000x reference)
import functools

import jax
import jax.numpy as jnp
from jax import lax
from jax.experimental import pallas as pl
from jax.experimental.pallas import tpu as pltpu
from jax.experimental.pallas import tpu_sc as plsc

N = 10000
E = 320000
F = 128
G = 512

NC = 2
NS = 16
NW = NC * NS

EPT = E // NW
K = 80
NCHUNK = EPT // K
DK = 80
ROW_CHUNKS = N // DK


def _zero_vmem_rows(buf, rows):
    z16 = jnp.zeros((16,), jnp.float32)

    def zr(r, c):
        def zc(j, c2):
            buf[r, pl.ds(j * 16, 16)] = z16
            return c2
        return lax.fori_loop(0, F // 16, zc, c)

    lax.fori_loop(0, rows, zr, 0)


def _segment_sum_sc(h, echunks):
    mesh = plsc.VectorSubcoreMesh(core_axis_name="c", subcore_axis_name="s")

    @functools.partial(
        pl.kernel,
        out_type=jax.ShapeDtypeStruct((NC, N, F), jnp.float32),
        mesh=mesh,
        scratch_types=[
            [pltpu.VMEM((2, K), jnp.int32)] * 4,
            [pltpu.VMEM((K,), jnp.int32)] * 2,
            [pltpu.VMEM((K, F), jnp.float32)] * 4,
            pltpu.VMEM_SHARED((N, F), jnp.float32),
            [pltpu.SemaphoreType.DMA] * 4,
            [pltpu.SemaphoreType.DMA] * 4,
            [pltpu.SemaphoreType.DMA] * 2,
        ],
    )
    def seg_sum(h_hbm, e_hbm, out_hbm, eidx, didx, rows, agg_sh, isem, gsem,
                ssem):
        cid = lax.axis_index("c")
        sub = lax.axis_index("s")
        wid = sub * NC + cid
        cbase = wid * NCHUNK
        rows0 = rows[0]

        _zero_vmem_rows(rows0, DK)

        def zchunk(j, c):
            i = sub + NS * j

            @pl.when(i < ROW_CHUNKS)
            def _():
                pltpu.sync_copy(rows0.at[pl.ds(0, DK)],
                                agg_sh.at[pl.ds(i * DK, DK)])
            return c

        lax.fori_loop(0, (ROW_CHUNKS + NS - 1) // NS, zchunk, 0)
        plsc.subcore_barrier()

        for p in range(4):
            pltpu.async_copy(e_hbm.at[cbase + p], eidx[p], isem[p])
        pltpu.make_async_copy(e_hbm.at[cbase], eidx[0], isem[0]).wait()
        pltpu.async_copy(h_hbm.at[eidx[0].at[0]], rows[0], gsem[0])

        def chunk_body(i, s4, s2):
            t4 = (s4 + 1) % 4
            pltpu.make_async_copy(h_hbm.at[eidx[s4].at[0]], rows[s4],
                                  gsem[s4]).wait()

            @pl.when(i >= 2)
            def _():
                pltpu.make_async_copy(rows[(s4 + 2) % 4],
                                      agg_sh.at[didx[s2]], ssem[s2]).wait()

            for j in range(K // 16):
                didx[s2][pl.ds(j * 16, 16)] = eidx[s4][1, pl.ds(j * 16, 16)]
            pltpu.async_copy(rows[s4], agg_sh.at[didx[s2]], ssem[s2],
                             add=True)

            @pl.when(i + 1 < NCHUNK)
            def _():
                pltpu.make_async_copy(e_hbm.at[cbase + i + 1], eidx[t4],
                                      isem[t4]).wait()
                pltpu.async_copy(h_hbm.at[eidx[t4].at[0]], rows[t4], gsem[t4])

            @pl.when(i + 4 < NCHUNK)
            def _():
                pltpu.async_copy(e_hbm.at[cbase + i + 4], eidx[s4], isem[s4])

        def four_chunks(j, c):
            i0 = 4 * j
            chunk_body(i0, 0, 0)
            for b in range(1, 4):
                @pl.when(i0 + b < NCHUNK)
                def _(b=b):
                    chunk_body(i0 + b, b % 4, b % 2)
            return c

        lax.fori_loop(0, (NCHUNK + 3) // 4, four_chunks, 0)
        pltpu.make_async_copy(rows[(NCHUNK - 2) % 4],
                              agg_sh.at[didx[(NCHUNK - 2) % 2]],
                              ssem[(NCHUNK - 2) % 2]).wait()
        pltpu.make_async_copy(rows[(NCHUNK - 1) % 4],
                              agg_sh.at[didx[(NCHUNK - 1) % 2]],
                              ssem[(NCHUNK - 1) % 2]).wait()
        plsc.subcore_barrier()

        def dchunk(j, c):
            i = sub + NS * j

            @pl.when(i < ROW_CHUNKS)
            def _():
                pltpu.sync_copy(agg_sh.at[pl.ds(i * DK, DK)],
                                rows0.at[pl.ds(0, DK)])
                pltpu.sync_copy(rows0.at[pl.ds(0, DK)],
                                out_hbm.at[cid].at[pl.ds(i * DK, DK)])
            return c

        lax.fori_loop(0, (ROW_CHUNKS + NS - 1) // NS, dchunk, 0)

    return seg_sum(h, echunks)


def _fill_vmem_rows(buf, rows, value):
    v16 = jnp.full((16,), value, jnp.float32)

    def zr(r, c):
        def zc(j, c2):
            buf[r, pl.ds(j * 16, 16)] = v16
            return c2
        return lax.fori_loop(0, F // 16, zc, c)

    lax.fori_loop(0, rows, zr, 0)


def _pool_sc(h, batch2d):
    mesh = plsc.VectorSubcoreMesh(core_axis_name="c", subcore_axis_name="s")

    @functools.partial(
        pl.kernel,
        out_type=(
            jax.ShapeDtypeStruct((NC, G, F), jnp.float32),
            jax.ShapeDtypeStruct((NC, G, F), jnp.float32),
        ),
        mesh=mesh,
        scratch_types=[
            pltpu.VMEM((1, DK), jnp.int32),
            pltpu.VMEM((DK,), jnp.int32),
            pltpu.VMEM((DK, F), jnp.float32),
            pltpu.VMEM((DK, F), jnp.float32),
            pltpu.VMEM_SHARED((G, F), jnp.float32),
            pltpu.VMEM_SHARED((G, F), jnp.float32),
        ],
    )
    def pool(h_hbm, b_hbm, pool_hbm, cnt_hbm, bidx, bidx1d, rows, ones_r,
             pool_sh, cnt_sh):
        cid = lax.axis_index("c")
        sub = lax.axis_index("s")
        wid = sub * NC + cid

        _zero_vmem_rows(rows, DK)
        _fill_vmem_rows(ones_r, DK, 1.0)

        @pl.when(sub < 8)
        def _():
            pltpu.sync_copy(rows.at[pl.ds(0, 64)], pool_sh.at[pl.ds(sub * 64, 64)])

        @pl.when(sub >= 8)
        def _():
            pltpu.sync_copy(rows.at[pl.ds(0, 64)],
                            cnt_sh.at[pl.ds((sub - 8) * 64, 64)])

        plsc.subcore_barrier()

        def kchunk(j, c):
            i = wid + NW * j

            @pl.when(i < ROW_CHUNKS)
            def _():
                pltpu.sync_copy(h_hbm.at[pl.ds(i * DK, DK)], rows)
                pltpu.sync_copy(b_hbm.at[i], bidx)
                for j in range(DK // 16):
                    bidx1d[pl.ds(j * 16, 16)] = bidx[0, pl.ds(j * 16, 16)]
                pltpu.sync_copy(rows, pool_sh.at[bidx1d], add=True)
                pltpu.sync_copy(ones_r, cnt_sh.at[bidx1d], add=True)
            return c

        lax.fori_loop(0, (ROW_CHUNKS + NW - 1) // NW, kchunk, 0)
        plsc.subcore_barrier()

        @pl.when(sub < 8)
        def _():
            pltpu.sync_copy(pool_sh.at[pl.ds(sub * 64, 64)], rows.at[pl.ds(0, 64)])
            pltpu.sync_copy(rows.at[pl.ds(0, 64)],
                            pool_hbm.at[cid].at[pl.ds(sub * 64, 64)])

        @pl.when(sub >= 8)
        def _():
            pltpu.sync_copy(cnt_sh.at[pl.ds((sub - 8) * 64, 64)],
                            rows.at[pl.ds(0, 64)])
            pltpu.sync_copy(rows.at[pl.ds(0, 64)],
                            cnt_hbm.at[cid].at[pl.ds((sub - 8) * 64, 64)])

    return pool(h, batch2d)


def _gin_mlp_tc(h, p0, p1, W1, b1, W2, b2):
    BLK = 400
    grid = (N // BLK,)

    def body(h_ref, p0_ref, p1_ref, w1_ref, b1_ref, w2_ref, b2_ref, o_ref):
        hh = h_ref[...] + p0_ref[...] + p1_ref[...]
        z = jnp.dot(hh, w1_ref[...], preferred_element_type=jnp.float32, precision=lax.Precision.HIGHEST)
        z = jnp.maximum(z + b1_ref[...], 0.0)
        o = jnp.dot(z, w2_ref[...], preferred_element_type=jnp.float32, precision=lax.Precision.HIGHEST)
        o_ref[...] = jnp.maximum(o + b2_ref[...], 0.0)

    return pl.pallas_call(
        body,
        grid=grid,
        in_specs=[
            pl.BlockSpec((BLK, F), lambda i: (i, 0)),
            pl.BlockSpec((BLK, F), lambda i: (i, 0)),
            pl.BlockSpec((BLK, F), lambda i: (i, 0)),
            pl.BlockSpec((F, F), lambda i: (0, 0)),
            pl.BlockSpec((1, F), lambda i: (0, 0)),
            pl.BlockSpec((F, F), lambda i: (0, 0)),
            pl.BlockSpec((1, F), lambda i: (0, 0)),
        ],
        out_specs=pl.BlockSpec((BLK, F), lambda i: (i, 0)),
        out_shape=jax.ShapeDtypeStruct((N, F), jnp.float32),
    )(h, p0, p1, W1, b1.reshape(1, F), W2, b2.reshape(1, F))


def _head_tc(p0, p1, c0, c1, W1, b1, W2, b2):
    C = b2.shape[0]

    def body(p0_ref, p1_ref, c0_ref, c1_ref, w1_ref, b1_ref, w2_ref, b2_ref,
             o_ref):
        cnt = jnp.maximum(c0_ref[...] + c1_ref[...], 1.0)
        hg = (p0_ref[...] + p1_ref[...]) / cnt
        z = jnp.dot(hg, w1_ref[...], preferred_element_type=jnp.float32, precision=lax.Precision.HIGHEST)
        z = jnp.maximum(z + b1_ref[...], 0.0)
        o_ref[...] = (jnp.dot(z, w2_ref[...], preferred_element_type=jnp.float32, precision=lax.Precision.HIGHEST)
                      + b2_ref[...])

    return pl.pallas_call(
        body,
        out_shape=jax.ShapeDtypeStruct((G, C), jnp.float32),
    )(p0, p1, c0, c1, W1, b1.reshape(1, F), W2, b2.reshape(1, C))


def kernel(x, edge_index, batch,
           gnn0_W1, gnn0_b1, gnn0_W2, gnn0_b2,
           gnn1_W1, gnn1_b1, gnn1_W2, gnn1_b2,
           gnn2_W1, gnn2_b1, gnn2_W2, gnn2_b2,
           gnn3_W1, gnn3_b1, gnn3_W2, gnn3_b2,
           mlp_W1, mlp_b1, mlp_W2, mlp_b2):
    echunks = (edge_index.reshape(2, NW, NCHUNK, K)
               .transpose(1, 2, 0, 3)
               .reshape(NW * NCHUNK, 2, K))
    layers = [
        (gnn0_W1, gnn0_b1, gnn0_W2, gnn0_b2),
        (gnn1_W1, gnn1_b1, gnn1_W2, gnn1_b2),
        (gnn2_W1, gnn2_b1, gnn2_W2, gnn2_b2),
        (gnn3_W1, gnn3_b1, gnn3_W2, gnn3_b2),
    ]
    h = x
    for (W1, b1, W2, b2) in layers:
        parts = _segment_sum_sc(h, echunks)
        h = _gin_mlp_tc(h, parts[0], parts[1], W1, b1, W2, b2)
    pool_parts, cnt_parts = _pool_sc(h, batch.reshape(ROW_CHUNKS, 1, DK))
    return _head_tc(pool_parts[0], pool_parts[1], cnt_parts[0], cnt_parts[1],
                    mlp_W1, mlp_b1, mlp_W2, mlp_b2)


# speedup vs baseline: 1.3905x; 1.3905x over previous
import functools

import jax
import jax.numpy as jnp
from jax import lax
from jax.experimental import pallas as pl
from jax.experimental.pallas import tpu as pltpu
from jax.experimental.pallas import tpu_sc as plsc

N = 10000
E = 320000
F = 128
G = 512

NC = 2
NS = 16
NW = NC * NS

EPT = E // NW
K = 80
NCHUNK = EPT // K
DK = 80
ROW_CHUNKS = N // DK


def _zero_vmem_rows(buf, rows):
    z16 = jnp.zeros((16,), jnp.float32)

    def zr(r, c):
        def zc(j, c2):
            buf[r, pl.ds(j * 16, 16)] = z16
            return c2
        return lax.fori_loop(0, F // 16, zc, c)

    lax.fori_loop(0, rows, zr, 0)


def _segment_sum_sc(h, echunks):
    mesh = plsc.VectorSubcoreMesh(core_axis_name="c", subcore_axis_name="s")

    @functools.partial(
        pl.kernel,
        out_type=jax.ShapeDtypeStruct((NC, N, F), jnp.float32),
        mesh=mesh,
        scratch_types=[
            [pltpu.VMEM((2, K), jnp.int32)] * 4,
            [pltpu.VMEM((K,), jnp.int32)] * 2,
            [pltpu.VMEM((K, F), jnp.float32)] * 4,
            pltpu.VMEM_SHARED((N, F), jnp.float32),
            [pltpu.SemaphoreType.DMA] * 4,
            [pltpu.SemaphoreType.DMA] * 4,
            [pltpu.SemaphoreType.DMA] * 2,
        ],
    )
    def seg_sum(h_hbm, e_hbm, out_hbm, eidx, didx, rows, agg_sh, isem, gsem,
                ssem):
        cid = lax.axis_index("c")
        sub = lax.axis_index("s")
        wid = sub * NC + cid
        cbase = wid * NCHUNK
        rows0 = rows[0]

        _zero_vmem_rows(rows0, DK)

        def zchunk(j, c):
            i = sub + NS * j

            @pl.when(i < ROW_CHUNKS)
            def _():
                pltpu.sync_copy(rows0.at[pl.ds(0, DK)],
                                agg_sh.at[pl.ds(i * DK, DK)])
            return c

        lax.fori_loop(0, (ROW_CHUNKS + NS - 1) // NS, zchunk, 0)
        plsc.subcore_barrier()

        for p in range(4):
            pltpu.async_copy(e_hbm.at[cbase + p], eidx[p], isem[p])
        pltpu.make_async_copy(e_hbm.at[cbase], eidx[0], isem[0]).wait()
        pltpu.async_copy(h_hbm.at[eidx[0].at[0]], rows[0], gsem[0])

        def chunk_body(i, s4, s2):
            t4 = (s4 + 1) % 4

            @pl.when(i >= 2)
            def _():
                pltpu.make_async_copy(rows[(s4 + 2) % 4],
                                      agg_sh.at[didx[s2]], ssem[s2]).wait()

            @pl.when(i + 1 < NCHUNK)
            def _():
                pltpu.make_async_copy(e_hbm.at[cbase + i + 1], eidx[t4],
                                      isem[t4]).wait()
                pltpu.async_copy(h_hbm.at[eidx[t4].at[0]], rows[t4], gsem[t4])

            pltpu.make_async_copy(h_hbm.at[eidx[s4].at[0]], rows[s4],
                                  gsem[s4]).wait()

            for j in range(K // 16):
                didx[s2][pl.ds(j * 16, 16)] = eidx[s4][1, pl.ds(j * 16, 16)]
            pltpu.async_copy(rows[s4], agg_sh.at[didx[s2]], ssem[s2],
                             add=True)

            @pl.when(i + 4 < NCHUNK)
            def _():
                pltpu.async_copy(e_hbm.at[cbase + i + 4], eidx[s4], isem[s4])

        def four_chunks(j, c):
            i0 = 4 * j
            chunk_body(i0, 0, 0)
            for b in range(1, 4):
                @pl.when(i0 + b < NCHUNK)
                def _(b=b):
                    chunk_body(i0 + b, b % 4, b % 2)
            return c

        lax.fori_loop(0, (NCHUNK + 3) // 4, four_chunks, 0)
        pltpu.make_async_copy(rows[(NCHUNK - 2) % 4],
                              agg_sh.at[didx[(NCHUNK - 2) % 2]],
                              ssem[(NCHUNK - 2) % 2]).wait()
        pltpu.make_async_copy(rows[(NCHUNK - 1) % 4],
                              agg_sh.at[didx[(NCHUNK - 1) % 2]],
                              ssem[(NCHUNK - 1) % 2]).wait()
        plsc.subcore_barrier()

        def dchunk(j, c):
            i = sub + NS * j

            @pl.when(i < ROW_CHUNKS)
            def _():
                pltpu.sync_copy(agg_sh.at[pl.ds(i * DK, DK)],
                                rows0.at[pl.ds(0, DK)])
                pltpu.sync_copy(rows0.at[pl.ds(0, DK)],
                                out_hbm.at[cid].at[pl.ds(i * DK, DK)])
            return c

        lax.fori_loop(0, (ROW_CHUNKS + NS - 1) // NS, dchunk, 0)

    return seg_sum(h, echunks)


def _fill_vmem_rows(buf, rows, value):
    v16 = jnp.full((16,), value, jnp.float32)

    def zr(r, c):
        def zc(j, c2):
            buf[r, pl.ds(j * 16, 16)] = v16
            return c2
        return lax.fori_loop(0, F // 16, zc, c)

    lax.fori_loop(0, rows, zr, 0)


def _pool_sc(h, batch2d):
    mesh = plsc.VectorSubcoreMesh(core_axis_name="c", subcore_axis_name="s")

    @functools.partial(
        pl.kernel,
        out_type=(
            jax.ShapeDtypeStruct((NC, G, F), jnp.float32),
            jax.ShapeDtypeStruct((NC, G, F), jnp.float32),
        ),
        mesh=mesh,
        scratch_types=[
            pltpu.VMEM((1, DK), jnp.int32),
            pltpu.VMEM((DK,), jnp.int32),
            pltpu.VMEM((DK, F), jnp.float32),
            pltpu.VMEM((DK, F), jnp.float32),
            pltpu.VMEM_SHARED((G, F), jnp.float32),
            pltpu.VMEM_SHARED((G, F), jnp.float32),
        ],
    )
    def pool(h_hbm, b_hbm, pool_hbm, cnt_hbm, bidx, bidx1d, rows, ones_r,
             pool_sh, cnt_sh):
        cid = lax.axis_index("c")
        sub = lax.axis_index("s")
        wid = sub * NC + cid

        _zero_vmem_rows(rows, DK)
        _fill_vmem_rows(ones_r, DK, 1.0)

        @pl.when(sub < 8)
        def _():
            pltpu.sync_copy(rows.at[pl.ds(0, 64)], pool_sh.at[pl.ds(sub * 64, 64)])

        @pl.when(sub >= 8)
        def _():
            pltpu.sync_copy(rows.at[pl.ds(0, 64)],
                            cnt_sh.at[pl.ds((sub - 8) * 64, 64)])

        plsc.subcore_barrier()

        def kchunk(j, c):
            i = wid + NW * j

            @pl.when(i < ROW_CHUNKS)
            def _():
                pltpu.sync_copy(h_hbm.at[pl.ds(i * DK, DK)], rows)
                pltpu.sync_copy(b_hbm.at[i], bidx)
                for j in range(DK // 16):
                    bidx1d[pl.ds(j * 16, 16)] = bidx[0, pl.ds(j * 16, 16)]
                pltpu.sync_copy(rows, pool_sh.at[bidx1d], add=True)
                pltpu.sync_copy(ones_r, cnt_sh.at[bidx1d], add=True)
            return c

        lax.fori_loop(0, (ROW_CHUNKS + NW - 1) // NW, kchunk, 0)
        plsc.subcore_barrier()

        @pl.when(sub < 8)
        def _():
            pltpu.sync_copy(pool_sh.at[pl.ds(sub * 64, 64)], rows.at[pl.ds(0, 64)])
            pltpu.sync_copy(rows.at[pl.ds(0, 64)],
                            pool_hbm.at[cid].at[pl.ds(sub * 64, 64)])

        @pl.when(sub >= 8)
        def _():
            pltpu.sync_copy(cnt_sh.at[pl.ds((sub - 8) * 64, 64)],
                            rows.at[pl.ds(0, 64)])
            pltpu.sync_copy(rows.at[pl.ds(0, 64)],
                            cnt_hbm.at[cid].at[pl.ds((sub - 8) * 64, 64)])

    return pool(h, batch2d)


def _gin_mlp_tc(h, p0, p1, W1, b1, W2, b2):
    BLK = 400
    grid = (N // BLK,)

    def body(h_ref, p0_ref, p1_ref, w1_ref, b1_ref, w2_ref, b2_ref, o_ref):
        hh = h_ref[...] + p0_ref[...] + p1_ref[...]
        z = jnp.dot(hh, w1_ref[...], preferred_element_type=jnp.float32, precision=lax.Precision.HIGHEST)
        z = jnp.maximum(z + b1_ref[...], 0.0)
        o = jnp.dot(z, w2_ref[...], preferred_element_type=jnp.float32, precision=lax.Precision.HIGHEST)
        o_ref[...] = jnp.maximum(o + b2_ref[...], 0.0)

    return pl.pallas_call(
        body,
        grid=grid,
        in_specs=[
            pl.BlockSpec((BLK, F), lambda i: (i, 0)),
            pl.BlockSpec((BLK, F), lambda i: (i, 0)),
            pl.BlockSpec((BLK, F), lambda i: (i, 0)),
            pl.BlockSpec((F, F), lambda i: (0, 0)),
            pl.BlockSpec((1, F), lambda i: (0, 0)),
            pl.BlockSpec((F, F), lambda i: (0, 0)),
            pl.BlockSpec((1, F), lambda i: (0, 0)),
        ],
        out_specs=pl.BlockSpec((BLK, F), lambda i: (i, 0)),
        out_shape=jax.ShapeDtypeStruct((N, F), jnp.float32),
    )(h, p0, p1, W1, b1.reshape(1, F), W2, b2.reshape(1, F))


def _head_tc(p0, p1, c0, c1, W1, b1, W2, b2):
    C = b2.shape[0]

    def body(p0_ref, p1_ref, c0_ref, c1_ref, w1_ref, b1_ref, w2_ref, b2_ref,
             o_ref):
        cnt = jnp.maximum(c0_ref[...] + c1_ref[...], 1.0)
        hg = (p0_ref[...] + p1_ref[...]) / cnt
        z = jnp.dot(hg, w1_ref[...], preferred_element_type=jnp.float32, precision=lax.Precision.HIGHEST)
        z = jnp.maximum(z + b1_ref[...], 0.0)
        o_ref[...] = (jnp.dot(z, w2_ref[...], preferred_element_type=jnp.float32, precision=lax.Precision.HIGHEST)
                      + b2_ref[...])

    return pl.pallas_call(
        body,
        out_shape=jax.ShapeDtypeStruct((G, C), jnp.float32),
    )(p0, p1, c0, c1, W1, b1.reshape(1, F), W2, b2.reshape(1, C))


def kernel(x, edge_index, batch,
           gnn0_W1, gnn0_b1, gnn0_W2, gnn0_b2,
           gnn1_W1, gnn1_b1, gnn1_W2, gnn1_b2,
           gnn2_W1, gnn2_b1, gnn2_W2, gnn2_b2,
           gnn3_W1, gnn3_b1, gnn3_W2, gnn3_b2,
           mlp_W1, mlp_b1, mlp_W2, mlp_b2):
    echunks = (edge_index.reshape(2, NW, NCHUNK, K)
               .transpose(1, 2, 0, 3)
               .reshape(NW * NCHUNK, 2, K))
    layers = [
        (gnn0_W1, gnn0_b1, gnn0_W2, gnn0_b2),
        (gnn1_W1, gnn1_b1, gnn1_W2, gnn1_b2),
        (gnn2_W1, gnn2_b1, gnn2_W2, gnn2_b2),
        (gnn3_W1, gnn3_b1, gnn3_W2, gnn3_b2),
    ]
    h = x
    for (W1, b1, W2, b2) in layers:
        parts = _segment_sum_sc(h, echunks)
        h = _gin_mlp_tc(h, parts[0], parts[1], W1, b1, W2, b2)
    pool_parts, cnt_parts = _pool_sc(h, batch.reshape(ROW_CHUNKS, 1, DK))
    return _head_tc(pool_parts[0], pool_parts[1], cnt_parts[0], cnt_parts[1],
                    mlp_W1, mlp_b1, mlp_W2, mlp_b2)


# speedup vs baseline: 1.4615x; 1.0511x over previous
import functools

import jax
import jax.numpy as jnp
from jax import lax
from jax.experimental import pallas as pl
from jax.experimental.pallas import tpu as pltpu
from jax.experimental.pallas import tpu_sc as plsc

N = 10000
E = 320000
F = 128
G = 512

NC = 2
NS = 16
NW = NC * NS

EPT = E // NW
K = 80
NCHUNK = EPT // K
DK = 80
ROW_CHUNKS = N // DK


def _zero_vmem_rows(buf, rows):
    z16 = jnp.zeros((16,), jnp.float32)

    def zr(r, c):
        def zc(j, c2):
            buf[r, pl.ds(j * 16, 16)] = z16
            return c2
        return lax.fori_loop(0, F // 16, zc, c)

    lax.fori_loop(0, rows, zr, 0)


def _segment_sum_sc(h, echunks):
    mesh = plsc.VectorSubcoreMesh(core_axis_name="c", subcore_axis_name="s")

    @functools.partial(
        pl.kernel,
        out_type=jax.ShapeDtypeStruct((NC, N, F), jnp.float32),
        mesh=mesh,
        scratch_types=[
            [pltpu.VMEM((2, K), jnp.int32)] * 4,
            [pltpu.VMEM((K,), jnp.int32)] * 2,
            [pltpu.VMEM((K, F), jnp.float32)] * 4,
            pltpu.VMEM_SHARED((N, F), jnp.float32),
            [pltpu.SemaphoreType.DMA] * 4,
            [pltpu.SemaphoreType.DMA] * 4,
            [pltpu.SemaphoreType.DMA] * 2,
        ],
    )
    def seg_sum(h_hbm, e_hbm, out_hbm, eidx, didx, rows, agg_sh, isem, gsem,
                ssem):
        cid = lax.axis_index("c")
        sub = lax.axis_index("s")
        wid = sub * NC + cid
        cbase = wid * NCHUNK
        rows0 = rows[0]

        _zero_vmem_rows(rows0, DK)

        def zchunk(j, c):
            i = sub + NS * j

            @pl.when(i < ROW_CHUNKS)
            def _():
                pltpu.sync_copy(rows0.at[pl.ds(0, DK)],
                                agg_sh.at[pl.ds(i * DK, DK)])
            return c

        lax.fori_loop(0, (ROW_CHUNKS + NS - 1) // NS, zchunk, 0)
        plsc.subcore_barrier()

        for p in range(4):
            pltpu.async_copy(e_hbm.at[cbase + p], eidx[p], isem[p])
        for p in range(2):
            pltpu.make_async_copy(e_hbm.at[cbase + p], eidx[p], isem[p]).wait()
            pltpu.async_copy(h_hbm.at[eidx[p].at[0]], rows[p], gsem[p])

        def chunk_body(i, s4, s2):
            t4 = (s4 + 2) % 4

            @pl.when(i >= 2)
            def _():
                pltpu.make_async_copy(rows[(s4 + 2) % 4],
                                      agg_sh.at[didx[s2]], ssem[s2]).wait()

            @pl.when(i + 2 < NCHUNK)
            def _():
                pltpu.make_async_copy(e_hbm.at[cbase + i + 2], eidx[t4],
                                      isem[t4]).wait()
                pltpu.async_copy(h_hbm.at[eidx[t4].at[0]], rows[t4], gsem[t4])

            pltpu.make_async_copy(h_hbm.at[eidx[s4].at[0]], rows[s4],
                                  gsem[s4]).wait()

            for j in range(K // 16):
                didx[s2][pl.ds(j * 16, 16)] = eidx[s4][1, pl.ds(j * 16, 16)]
            pltpu.async_copy(rows[s4], agg_sh.at[didx[s2]], ssem[s2],
                             add=True)

            @pl.when(i + 4 < NCHUNK)
            def _():
                pltpu.async_copy(e_hbm.at[cbase + i + 4], eidx[s4], isem[s4])

        def four_chunks(j, c):
            i0 = 4 * j
            chunk_body(i0, 0, 0)
            for b in range(1, 4):
                @pl.when(i0 + b < NCHUNK)
                def _(b=b):
                    chunk_body(i0 + b, b % 4, b % 2)
            return c

        lax.fori_loop(0, (NCHUNK + 3) // 4, four_chunks, 0)
        pltpu.make_async_copy(rows[(NCHUNK - 2) % 4],
                              agg_sh.at[didx[(NCHUNK - 2) % 2]],
                              ssem[(NCHUNK - 2) % 2]).wait()
        pltpu.make_async_copy(rows[(NCHUNK - 1) % 4],
                              agg_sh.at[didx[(NCHUNK - 1) % 2]],
                              ssem[(NCHUNK - 1) % 2]).wait()
        plsc.subcore_barrier()

        def dchunk(j, c):
            i = sub + NS * j

            @pl.when(i < ROW_CHUNKS)
            def _():
                pltpu.sync_copy(agg_sh.at[pl.ds(i * DK, DK)],
                                rows0.at[pl.ds(0, DK)])
                pltpu.sync_copy(rows0.at[pl.ds(0, DK)],
                                out_hbm.at[cid].at[pl.ds(i * DK, DK)])
            return c

        lax.fori_loop(0, (ROW_CHUNKS + NS - 1) // NS, dchunk, 0)

    return seg_sum(h, echunks)


def _fill_vmem_rows(buf, rows, value):
    v16 = jnp.full((16,), value, jnp.float32)

    def zr(r, c):
        def zc(j, c2):
            buf[r, pl.ds(j * 16, 16)] = v16
            return c2
        return lax.fori_loop(0, F // 16, zc, c)

    lax.fori_loop(0, rows, zr, 0)


def _pool_sc(h, batch2d):
    mesh = plsc.VectorSubcoreMesh(core_axis_name="c", subcore_axis_name="s")

    @functools.partial(
        pl.kernel,
        out_type=(
            jax.ShapeDtypeStruct((NC, G, F), jnp.float32),
            jax.ShapeDtypeStruct((NC, G, F), jnp.float32),
        ),
        mesh=mesh,
        scratch_types=[
            pltpu.VMEM((1, DK), jnp.int32),
            pltpu.VMEM((DK,), jnp.int32),
            pltpu.VMEM((DK, F), jnp.float32),
            pltpu.VMEM((DK, F), jnp.float32),
            pltpu.VMEM_SHARED((G, F), jnp.float32),
            pltpu.VMEM_SHARED((G, F), jnp.float32),
        ],
    )
    def pool(h_hbm, b_hbm, pool_hbm, cnt_hbm, bidx, bidx1d, rows, ones_r,
             pool_sh, cnt_sh):
        cid = lax.axis_index("c")
        sub = lax.axis_index("s")
        wid = sub * NC + cid

        _zero_vmem_rows(rows, DK)
        _fill_vmem_rows(ones_r, DK, 1.0)

        @pl.when(sub < 8)
        def _():
            pltpu.sync_copy(rows.at[pl.ds(0, 64)], pool_sh.at[pl.ds(sub * 64, 64)])

        @pl.when(sub >= 8)
        def _():
            pltpu.sync_copy(rows.at[pl.ds(0, 64)],
                            cnt_sh.at[pl.ds((sub - 8) * 64, 64)])

        plsc.subcore_barrier()

        def kchunk(j, c):
            i = wid + NW * j

            @pl.when(i < ROW_CHUNKS)
            def _():
                pltpu.sync_copy(h_hbm.at[pl.ds(i * DK, DK)], rows)
                pltpu.sync_copy(b_hbm.at[i], bidx)
                for j in range(DK // 16):
                    bidx1d[pl.ds(j * 16, 16)] = bidx[0, pl.ds(j * 16, 16)]
                pltpu.sync_copy(rows, pool_sh.at[bidx1d], add=True)
                pltpu.sync_copy(ones_r, cnt_sh.at[bidx1d], add=True)
            return c

        lax.fori_loop(0, (ROW_CHUNKS + NW - 1) // NW, kchunk, 0)
        plsc.subcore_barrier()

        @pl.when(sub < 8)
        def _():
            pltpu.sync_copy(pool_sh.at[pl.ds(sub * 64, 64)], rows.at[pl.ds(0, 64)])
            pltpu.sync_copy(rows.at[pl.ds(0, 64)],
                            pool_hbm.at[cid].at[pl.ds(sub * 64, 64)])

        @pl.when(sub >= 8)
        def _():
            pltpu.sync_copy(cnt_sh.at[pl.ds((sub - 8) * 64, 64)],
                            rows.at[pl.ds(0, 64)])
            pltpu.sync_copy(rows.at[pl.ds(0, 64)],
                            cnt_hbm.at[cid].at[pl.ds((sub - 8) * 64, 64)])

    return pool(h, batch2d)


def _gin_mlp_tc(h, p0, p1, W1, b1, W2, b2):
    BLK = 400
    grid = (N // BLK,)

    def body(h_ref, p0_ref, p1_ref, w1_ref, b1_ref, w2_ref, b2_ref, o_ref):
        hh = h_ref[...] + p0_ref[...] + p1_ref[...]
        z = jnp.dot(hh, w1_ref[...], preferred_element_type=jnp.float32)
        z = jnp.maximum(z + b1_ref[...], 0.0)
        o = jnp.dot(z, w2_ref[...], preferred_element_type=jnp.float32)
        o_ref[...] = jnp.maximum(o + b2_ref[...], 0.0)

    return pl.pallas_call(
        body,
        grid=grid,
        in_specs=[
            pl.BlockSpec((BLK, F), lambda i: (i, 0)),
            pl.BlockSpec((BLK, F), lambda i: (i, 0)),
            pl.BlockSpec((BLK, F), lambda i: (i, 0)),
            pl.BlockSpec((F, F), lambda i: (0, 0)),
            pl.BlockSpec((1, F), lambda i: (0, 0)),
            pl.BlockSpec((F, F), lambda i: (0, 0)),
            pl.BlockSpec((1, F), lambda i: (0, 0)),
        ],
        out_specs=pl.BlockSpec((BLK, F), lambda i: (i, 0)),
        out_shape=jax.ShapeDtypeStruct((N, F), jnp.float32),
    )(h, p0, p1, W1, b1.reshape(1, F), W2, b2.reshape(1, F))


def _head_tc(p0, p1, c0, c1, W1, b1, W2, b2):
    C = b2.shape[0]

    def body(p0_ref, p1_ref, c0_ref, c1_ref, w1_ref, b1_ref, w2_ref, b2_ref,
             o_ref):
        cnt = jnp.maximum(c0_ref[...] + c1_ref[...], 1.0)
        hg = (p0_ref[...] + p1_ref[...]) / cnt
        z = jnp.dot(hg, w1_ref[...], preferred_element_type=jnp.float32)
        z = jnp.maximum(z + b1_ref[...], 0.0)
        o_ref[...] = (jnp.dot(z, w2_ref[...], preferred_element_type=jnp.float32)
                      + b2_ref[...])

    return pl.pallas_call(
        body,
        out_shape=jax.ShapeDtypeStruct((G, C), jnp.float32),
    )(p0, p1, c0, c1, W1, b1.reshape(1, F), W2, b2.reshape(1, C))


def kernel(x, edge_index, batch,
           gnn0_W1, gnn0_b1, gnn0_W2, gnn0_b2,
           gnn1_W1, gnn1_b1, gnn1_W2, gnn1_b2,
           gnn2_W1, gnn2_b1, gnn2_W2, gnn2_b2,
           gnn3_W1, gnn3_b1, gnn3_W2, gnn3_b2,
           mlp_W1, mlp_b1, mlp_W2, mlp_b2):
    echunks = (edge_index.reshape(2, NW, NCHUNK, K)
               .transpose(1, 2, 0, 3)
               .reshape(NW * NCHUNK, 2, K))
    layers = [
        (gnn0_W1, gnn0_b1, gnn0_W2, gnn0_b2),
        (gnn1_W1, gnn1_b1, gnn1_W2, gnn1_b2),
        (gnn2_W1, gnn2_b1, gnn2_W2, gnn2_b2),
        (gnn3_W1, gnn3_b1, gnn3_W2, gnn3_b2),
    ]
    h = x
    for (W1, b1, W2, b2) in layers:
        parts = _segment_sum_sc(h, echunks)
        h = _gin_mlp_tc(h, parts[0], parts[1], W1, b1, W2, b2)
    pool_parts, cnt_parts = _pool_sc(h, batch.reshape(ROW_CHUNKS, 1, DK))
    return _head_tc(pool_parts[0], pool_parts[1], cnt_parts[0], cnt_parts[1],
                    mlp_W1, mlp_b1, mlp_W2, mlp_b2)
